# baseline (speedup 1.0000x reference)
import functools

import numpy as np

import jax
import jax.numpy as jnp
from jax import lax
from jax.experimental import pallas as pl
from jax.experimental.pallas import tpu as pltpu
from jax.experimental.pallas import tpu_sc as plsc

NC = 2
NS = 16
LANES = 16


def _xm_body(x_ref, w_ref, o_ref):
    o_ref[...] = jnp.dot(x_ref[...], w_ref[...],
                         preferred_element_type=jnp.float32)


def _em_body(ea_t_ref, w_ref, b_ref, o_ref):
    res = lax.dot_general(
        ea_t_ref[...], w_ref[...], (((0,), (0,)), ((), ())),
        preferred_element_type=jnp.float32) + b_ref[...]
    half = res.shape[1] // 2
    lo = lax.bitcast_convert_type(
        res[:, :half].astype(jnp.bfloat16), jnp.uint16).astype(jnp.uint32)
    hi = lax.bitcast_convert_type(
        res[:, half:].astype(jnp.bfloat16), jnp.uint16).astype(jnp.uint32)
    o_ref[...] = lax.bitcast_convert_type(lo | (hi << 16), jnp.int32)


def _epilogue_body(p0_ref, p1_ref, x_ref, ws_ref, wg_ref, wv_ref, o_ref):
    h = p0_ref[0] + p1_ref[0] + jnp.dot(
        x_ref[...], ws_ref[...], preferred_element_type=jnp.float32)
    g = jnp.dot(h, wg_ref[...], preferred_element_type=jnp.float32)
    v = jnp.dot(h, wv_ref[...], preferred_element_type=jnp.float32)
    o_ref[...] = jax.nn.sigmoid(g) * jnp.tanh(v)


def _make_sc_agg(n_pad, n_edges, d_hid, chunk):
    edges_per_tile = n_edges // (NC * NS)
    n_chunks = edges_per_tile // chunk
    rows_per_tile = n_pad // NS
    zrows = rows_per_tile // 8
    n_zcopies = 8

    mesh = plsc.VectorSubcoreMesh(
        core_axis_name="c", subcore_axis_name="s",
        num_cores=NC, num_subcores=NS)

    @functools.partial(
        pl.kernel,
        mesh=mesh,
        out_type=jax.ShapeDtypeStruct((NC, n_pad, d_hid), jnp.float32),
        scratch_types=[
            pltpu.VMEM_SHARED((n_pad, d_hid), jnp.float32),
            pltpu.VMEM((chunk,), jnp.int32),
            pltpu.VMEM((chunk,), jnp.int32),
            pltpu.VMEM((chunk,), jnp.int32),
            pltpu.VMEM((chunk,), jnp.int32),
            pltpu.VMEM((2, chunk, d_hid), jnp.float32),
            pltpu.VMEM((2, chunk, d_hid // 2), jnp.int32),
            pltpu.SemaphoreType.DMA,
            pltpu.SemaphoreType.DMA,
        ],
    )
    def sc_agg(xm_hbm, em_hbm, ei_hbm, out_hbm,
               agg_sh, src0, src1, dst0, dst1, rows_v, em_v,
               gsem0, gsem1):
        c = lax.axis_index("c")
        s = lax.axis_index("s")
        wid = c * NS + s

        def zero_row(r, _):
            for j in range(d_hid // LANES):
                rows_v[0, r, pl.ds(j * LANES, LANES)] = jnp.zeros(
                    (LANES,), jnp.float32)
            return _
        lax.fori_loop(0, zrows, zero_row, 0)
        row0 = s * rows_per_tile
        for z in range(n_zcopies):
            pltpu.sync_copy(rows_v.at[0, pl.ds(0, zrows)],
                            agg_sh.at[pl.ds(row0 + z * zrows, zrows)])
        plsc.subcore_barrier()

        base_e = wid * edges_per_tile

        sems = (gsem0, gsem1)
        srcs = (src0, src1)
        dsts = (dst0, dst1)

        def start_fetch(k, b):
            pltpu.async_copy(xm_hbm.at[srcs[b]], rows_v.at[b], sems[b])
            eoff = pl.multiple_of(base_e + k * chunk, 8)
            pltpu.async_copy(em_hbm.at[pl.ds(eoff, chunk)], em_v.at[b],
                             sems[b])
            doff = pl.multiple_of(n_edges + base_e + k * chunk, 8)
            pltpu.async_copy(ei_hbm.at[pl.ds(doff, chunk)], dsts[b],
                             sems[b])

            @pl.when(k + 1 < n_chunks)
            def _():
                soff = pl.multiple_of(base_e + (k + 1) * chunk, 8)
                pltpu.async_copy(ei_hbm.at[pl.ds(soff, chunk)],
                                 srcs[1 - b], sems[b])

        def wait_fetch(k, b):
            pltpu.make_async_copy(xm_hbm.at[srcs[b]],
                                  rows_v.at[b], sems[b]).wait()
            pltpu.make_async_copy(em_hbm.at[pl.ds(base_e, chunk)],
                                  em_v.at[b], sems[b]).wait()
            pltpu.make_async_copy(ei_hbm.at[pl.ds(base_e, chunk)],
                                  dsts[b], sems[b]).wait()

            @pl.when(k + 1 < n_chunks)
            def _():
                pltpu.make_async_copy(ei_hbm.at[pl.ds(base_e, chunk)],
                                      srcs[1 - b], sems[b]).wait()

        def half_step(k, b):
            wait_fetch(k, b)

            @pl.when(k + 1 < n_chunks)
            def _():
                start_fetch(k + 1, 1 - b)

            nq = d_hid // (2 * LANES)

            def relu_row(r, carry):
                for q in range(nq):
                    w = em_v[b, r, pl.ds(q * LANES, LANES)]
                    lo = lax.bitcast_convert_type(w << 16, jnp.float32)
                    hi = lax.bitcast_convert_type(w & jnp.int32(-65536),
                                                  jnp.float32)
                    sl0 = pl.ds(q * LANES, LANES)
                    sl1 = pl.ds((q + nq) * LANES, LANES)
                    rows_v[b, r, sl0] = jnp.maximum(
                        rows_v[b, r, sl0] + lo, 0.0)
                    rows_v[b, r, sl1] = jnp.maximum(
                        rows_v[b, r, sl1] + hi, 0.0)
                return carry
            lax.fori_loop(0, chunk, relu_row, 0)

            pltpu.sync_copy(rows_v.at[b], agg_sh.at[dsts[b]],
                            add=True)

        pltpu.sync_copy(ei_hbm.at[pl.ds(base_e, chunk)], src0)
        start_fetch(0, 0)

        def pair_body(t, carry_in):
            half_step(2 * t, 0)
            half_step(2 * t + 1, 1)
            return carry_in
        lax.fori_loop(0, n_chunks // 2, pair_body, 0)
        if n_chunks % 2:
            half_step(n_chunks - 1, 0)

        plsc.subcore_barrier()
        pltpu.sync_copy(agg_sh.at[pl.ds(row0, rows_per_tile)],
                        out_hbm.at[c, pl.ds(row0, rows_per_tile)])

    return sc_agg


def kernel(x, edge_index, edge_attr, W_msg, W_edge, b_msg, W_self, W_gate,
           W_val):
    n_nodes, d_feat = x.shape
    n_edges = edge_index.shape[1]
    d_edge = edge_attr.shape[1]
    d_hid = W_msg.shape[1]
    d_out = W_gate.shape[1]

    ei_flat = edge_index.reshape(2 * n_edges)

    bn = 2000
    xm = pl.pallas_call(
        _xm_body,
        grid=(n_nodes // bn,),
        in_specs=[pl.BlockSpec((bn, d_feat), lambda i: (i, 0)),
                  pl.BlockSpec((d_feat, d_hid), lambda i: (0, 0))],
        out_specs=pl.BlockSpec((bn, d_hid), lambda i: (i, 0)),
        out_shape=jax.ShapeDtypeStruct((n_nodes, d_hid), jnp.float32),
    )(x, W_msg)

    be = 3200
    em = pl.pallas_call(
        _em_body,
        grid=(n_edges // be,),
        in_specs=[pl.BlockSpec((d_edge, be), lambda i: (0, i)),
                  pl.BlockSpec((d_edge, d_hid), lambda i: (0, 0)),
                  pl.BlockSpec((1, d_hid), lambda i: (0, 0))],
        out_specs=pl.BlockSpec((be, d_hid // 2), lambda i: (i, 0)),
        out_shape=jax.ShapeDtypeStruct((n_edges, d_hid // 2), jnp.int32),
    )(edge_attr.T, W_edge, b_msg.reshape(1, d_hid))

    n_pad = -(-n_nodes // (NS * 8)) * (NS * 8)
    sc_agg = _make_sc_agg(n_pad, n_edges, d_hid, chunk=80)
    partials = sc_agg(xm, em, ei_flat)

    out = pl.pallas_call(
        _epilogue_body,
        grid=(n_nodes // bn,),
        in_specs=[pl.BlockSpec((1, bn, d_hid), lambda i: (0, i, 0)),
                  pl.BlockSpec((1, bn, d_hid), lambda i: (1, i, 0)),
                  pl.BlockSpec((bn, d_feat), lambda i: (i, 0)),
                  pl.BlockSpec((d_feat, d_hid), lambda i: (0, 0)),
                  pl.BlockSpec((d_hid, d_out), lambda i: (0, 0)),
                  pl.BlockSpec((d_hid, d_out), lambda i: (0, 0))],
        out_specs=pl.BlockSpec((bn, d_out), lambda i: (i, 0)),
        out_shape=jax.ShapeDtypeStruct((n_nodes, d_out), jnp.float32),
    )(partials, partials, x, W_self, W_gate, W_val)
    return out


# speedup vs baseline: 1.0083x; 1.0083x over previous
import functools

import jax
import jax.numpy as jnp
from jax import lax
from jax.experimental import pallas as pl
from jax.experimental.pallas import tpu as pltpu
from jax.experimental.pallas import tpu_sc as plsc

NC = 2
NS = 16
LANES = 16


def _xm_body(x_ref, w_ref, o_ref):
    o_ref[...] = jnp.dot(x_ref[...], w_ref[...],
                         preferred_element_type=jnp.float32)


def _em_body(ea_t_ref, w_ref, b_ref, o_ref):
    o_ref[...] = lax.dot_general(
        ea_t_ref[...], w_ref[...], (((0,), (0,)), ((), ())),
        preferred_element_type=jnp.float32) + b_ref[...]


def _epilogue_body(p0_ref, p1_ref, x_ref, ws_ref, wg_ref, wv_ref, o_ref):
    h = p0_ref[0] + p1_ref[0] + jnp.dot(
        x_ref[...], ws_ref[...], preferred_element_type=jnp.float32)
    g = jnp.dot(h, wg_ref[...], preferred_element_type=jnp.float32)
    v = jnp.dot(h, wv_ref[...], preferred_element_type=jnp.float32)
    o_ref[...] = jax.nn.sigmoid(g) * jnp.tanh(v)


def _make_sc_agg(n_pad, n_edges, d_hid, chunk):
    edges_per_tile = n_edges // (NC * NS)
    n_chunks = edges_per_tile // chunk
    rows_per_tile = n_pad // NS
    zrows = rows_per_tile // 8
    n_zcopies = 8

    mesh = plsc.VectorSubcoreMesh(
        core_axis_name="c", subcore_axis_name="s",
        num_cores=NC, num_subcores=NS)

    @functools.partial(
        pl.kernel,
        mesh=mesh,
        out_type=jax.ShapeDtypeStruct((NC, n_pad, d_hid), jnp.float32),
        scratch_types=[
            pltpu.VMEM_SHARED((n_pad, d_hid), jnp.float32),
            pltpu.VMEM((chunk,), jnp.int32),
            pltpu.VMEM((chunk,), jnp.int32),
            pltpu.VMEM((chunk,), jnp.int32),
            pltpu.VMEM((chunk,), jnp.int32),
            pltpu.VMEM((2, chunk, d_hid), jnp.float32),
            pltpu.VMEM((2, chunk, d_hid), jnp.float32),
            pltpu.SemaphoreType.DMA,
            pltpu.SemaphoreType.DMA,
        ],
    )
    def sc_agg(xm_hbm, em_hbm, ei_hbm, out_hbm,
               agg_sh, src0, src1, dst0, dst1, rows_v, em_v,
               gsem0, gsem1):
        c = lax.axis_index("c")
        s = lax.axis_index("s")
        wid = c * NS + s

        def zero_row(r, _):
            for j in range(d_hid // LANES):
                em_v[0, r, pl.ds(j * LANES, LANES)] = jnp.zeros(
                    (LANES,), jnp.float32)
            return _
        lax.fori_loop(0, zrows, zero_row, 0)
        row0 = s * rows_per_tile
        for z in range(n_zcopies):
            pltpu.sync_copy(em_v.at[0, pl.ds(0, zrows)],
                            agg_sh.at[pl.ds(row0 + z * zrows, zrows)])
        plsc.subcore_barrier()

        base_e = wid * edges_per_tile

        sems = (gsem0, gsem1)
        srcs = (src0, src1)
        dsts = (dst0, dst1)

        def start_fetch(k, b):
            pltpu.async_copy(xm_hbm.at[srcs[b]], rows_v.at[b], sems[b])
            eoff = pl.multiple_of(base_e + k * chunk, 8)
            pltpu.async_copy(em_hbm.at[pl.ds(eoff, chunk)], em_v.at[b],
                             sems[b])
            doff = pl.multiple_of(n_edges + base_e + k * chunk, 8)
            pltpu.async_copy(ei_hbm.at[pl.ds(doff, chunk)], dsts[b],
                             sems[b])

            @pl.when(k + 1 < n_chunks)
            def _():
                soff = pl.multiple_of(base_e + (k + 1) * chunk, 8)
                pltpu.async_copy(ei_hbm.at[pl.ds(soff, chunk)],
                                 srcs[1 - b], sems[b])

        def wait_fetch(k, b):
            pltpu.make_async_copy(xm_hbm.at[srcs[b]],
                                  rows_v.at[b], sems[b]).wait()
            pltpu.make_async_copy(em_hbm.at[pl.ds(base_e, chunk)],
                                  em_v.at[b], sems[b]).wait()
            pltpu.make_async_copy(ei_hbm.at[pl.ds(base_e, chunk)],
                                  dsts[b], sems[b]).wait()

            @pl.when(k + 1 < n_chunks)
            def _():
                pltpu.make_async_copy(ei_hbm.at[pl.ds(base_e, chunk)],
                                      srcs[1 - b], sems[b]).wait()

        def half_step(k, b):
            wait_fetch(k, b)

            @pl.when(k + 1 < n_chunks)
            def _():
                start_fetch(k + 1, 1 - b)

            def relu_rows(r4, carry):
                for u in range(4):
                    r = r4 * 4 + u
                    for j in range(d_hid // LANES):
                        sl = pl.ds(j * LANES, LANES)
                        rows_v[b, r, sl] = jnp.maximum(
                            rows_v[b, r, sl] + em_v[b, r, sl], 0.0)
                return carry
            lax.fori_loop(0, chunk // 4, relu_rows, 0)

            pltpu.sync_copy(rows_v.at[b], agg_sh.at[dsts[b]],
                            add=True)

        pltpu.sync_copy(ei_hbm.at[pl.ds(base_e, chunk)], src0)
        start_fetch(0, 0)

        def pair_body(t, carry_in):
            half_step(2 * t, 0)
            half_step(2 * t + 1, 1)
            return carry_in
        lax.fori_loop(0, n_chunks // 2, pair_body, 0)
        if n_chunks % 2:
            half_step(n_chunks - 1, 0)

        plsc.subcore_barrier()
        pltpu.sync_copy(agg_sh.at[pl.ds(row0, rows_per_tile)],
                        out_hbm.at[c, pl.ds(row0, rows_per_tile)])

    return sc_agg


def kernel(x, edge_index, edge_attr, W_msg, W_edge, b_msg, W_self, W_gate,
           W_val):
    n_nodes, d_feat = x.shape
    n_edges = edge_index.shape[1]
    d_edge = edge_attr.shape[1]
    d_hid = W_msg.shape[1]
    d_out = W_gate.shape[1]

    ei_flat = edge_index.reshape(2 * n_edges)

    bn = 2000
    xm = pl.pallas_call(
        _xm_body,
        grid=(n_nodes // bn,),
        in_specs=[pl.BlockSpec((bn, d_feat), lambda i: (i, 0)),
                  pl.BlockSpec((d_feat, d_hid), lambda i: (0, 0))],
        out_specs=pl.BlockSpec((bn, d_hid), lambda i: (i, 0)),
        out_shape=jax.ShapeDtypeStruct((n_nodes, d_hid), jnp.float32),
    )(x, W_msg)

    be = 3200
    em = pl.pallas_call(
        _em_body,
        grid=(n_edges // be,),
        in_specs=[pl.BlockSpec((d_edge, be), lambda i: (0, i)),
                  pl.BlockSpec((d_edge, d_hid), lambda i: (0, 0)),
                  pl.BlockSpec((1, d_hid), lambda i: (0, 0))],
        out_specs=pl.BlockSpec((be, d_hid), lambda i: (i, 0)),
        out_shape=jax.ShapeDtypeStruct((n_edges, d_hid), jnp.float32),
    )(edge_attr.T, W_edge, b_msg.reshape(1, d_hid))

    n_pad = -(-n_nodes // (NS * 8)) * (NS * 8)
    sc_agg = _make_sc_agg(n_pad, n_edges, d_hid, chunk=80)
    partials = sc_agg(xm, em, ei_flat)

    out = pl.pallas_call(
        _epilogue_body,
        grid=(n_nodes // bn,),
        in_specs=[pl.BlockSpec((1, bn, d_hid), lambda i: (0, i, 0)),
                  pl.BlockSpec((1, bn, d_hid), lambda i: (1, i, 0)),
                  pl.BlockSpec((bn, d_feat), lambda i: (i, 0)),
                  pl.BlockSpec((d_feat, d_hid), lambda i: (0, 0)),
                  pl.BlockSpec((d_hid, d_out), lambda i: (0, 0)),
                  pl.BlockSpec((d_hid, d_out), lambda i: (0, 0))],
        out_specs=pl.BlockSpec((bn, d_out), lambda i: (i, 0)),
        out_shape=jax.ShapeDtypeStruct((n_nodes, d_out), jnp.float32),
    )(partials, partials, x, W_self, W_gate, W_val)
    return out


# speedup vs baseline: 1.0114x; 1.0031x over previous
import functools

import jax
import jax.numpy as jnp
from jax import lax
from jax.experimental import pallas as pl
from jax.experimental.pallas import tpu as pltpu
from jax.experimental.pallas import tpu_sc as plsc

NC = 2
NS = 16
LANES = 16


def _xm_body(x_ref, w_ref, o_ref):
    o_ref[...] = jnp.dot(x_ref[...], w_ref[...],
                         preferred_element_type=jnp.float32)


def _em_body(ea_t_ref, w_ref, b_ref, o_ref):
    o_ref[...] = lax.dot_general(
        ea_t_ref[...], w_ref[...], (((0,), (0,)), ((), ())),
        preferred_element_type=jnp.float32) + b_ref[...]


def _epilogue_body(p0_ref, p1_ref, x_ref, ws_ref, wg_ref, wv_ref, o_ref):
    h = p0_ref[0] + p1_ref[0] + jnp.dot(
        x_ref[...], ws_ref[...], preferred_element_type=jnp.float32)
    g = jnp.dot(h, wg_ref[...], preferred_element_type=jnp.float32)
    v = jnp.dot(h, wv_ref[...], preferred_element_type=jnp.float32)
    o_ref[...] = jax.nn.sigmoid(g) * jnp.tanh(v)


def _make_sc_agg(n_pad, n_edges, d_hid, chunk):
    edges_per_tile = n_edges // (NC * NS)
    n_chunks = edges_per_tile // chunk
    rows_per_tile = n_pad // NS
    zrows = rows_per_tile // 8
    n_zcopies = 8

    mesh = plsc.VectorSubcoreMesh(
        core_axis_name="c", subcore_axis_name="s",
        num_cores=NC, num_subcores=NS)

    @functools.partial(
        pl.kernel,
        mesh=mesh,
        out_type=jax.ShapeDtypeStruct((NC, n_pad, d_hid), jnp.float32),
        scratch_types=[
            pltpu.VMEM_SHARED((n_pad, d_hid), jnp.float32),
            pltpu.VMEM((chunk,), jnp.int32),
            pltpu.VMEM((chunk,), jnp.int32),
            pltpu.VMEM((chunk,), jnp.int32),
            pltpu.VMEM((chunk,), jnp.int32),
            pltpu.VMEM((2, chunk, d_hid), jnp.float32),
            pltpu.VMEM((2, chunk, d_hid), jnp.float32),
            pltpu.SemaphoreType.DMA,
            pltpu.SemaphoreType.DMA,
        ],
    )
    def sc_agg(xm_hbm, em_hbm, ei_hbm, out_hbm,
               agg_sh, src0, src1, dst0, dst1, rows_v, em_v,
               gsem0, gsem1):
        c = lax.axis_index("c")
        s = lax.axis_index("s")
        wid = c * NS + s

        def zero_row(r, _):
            for j in range(d_hid // LANES):
                em_v[0, r, pl.ds(j * LANES, LANES)] = jnp.zeros(
                    (LANES,), jnp.float32)
            return _
        lax.fori_loop(0, zrows, zero_row, 0)
        row0 = s * rows_per_tile
        for z in range(n_zcopies):
            pltpu.sync_copy(em_v.at[0, pl.ds(0, zrows)],
                            agg_sh.at[pl.ds(row0 + z * zrows, zrows)])
        plsc.subcore_barrier()

        base_e = wid * edges_per_tile

        sems = (gsem0, gsem1)
        srcs = (src0, src1)
        dsts = (dst0, dst1)

        def start_fetch(k, b):
            pltpu.async_copy(xm_hbm.at[srcs[b]], rows_v.at[b], sems[b])
            eoff = pl.multiple_of(base_e + k * chunk, 8)
            pltpu.async_copy(em_hbm.at[pl.ds(eoff, chunk)], em_v.at[b],
                             sems[b])
            doff = pl.multiple_of(n_edges + base_e + k * chunk, 8)
            pltpu.async_copy(ei_hbm.at[pl.ds(doff, chunk)], dsts[b],
                             sems[b])

            @pl.when(k + 1 < n_chunks)
            def _():
                soff = pl.multiple_of(base_e + (k + 1) * chunk, 8)
                pltpu.async_copy(ei_hbm.at[pl.ds(soff, chunk)],
                                 srcs[1 - b], sems[b])

        def wait_fetch(k, b):
            pltpu.make_async_copy(xm_hbm.at[srcs[b]],
                                  rows_v.at[b], sems[b]).wait()
            pltpu.make_async_copy(em_hbm.at[pl.ds(base_e, chunk)],
                                  em_v.at[b], sems[b]).wait()
            pltpu.make_async_copy(ei_hbm.at[pl.ds(base_e, chunk)],
                                  dsts[b], sems[b]).wait()

            @pl.when(k + 1 < n_chunks)
            def _():
                pltpu.make_async_copy(ei_hbm.at[pl.ds(base_e, chunk)],
                                      srcs[1 - b], sems[b]).wait()

        def half_step(k, b):
            wait_fetch(k, b)

            @pl.when(k + 1 < n_chunks)
            def _():
                start_fetch(k + 1, 1 - b)

            def relu_row(r, carry):
                for j in range(d_hid // LANES):
                    sl = pl.ds(j * LANES, LANES)
                    rows_v[b, r, sl] = jnp.maximum(
                        rows_v[b, r, sl] + em_v[b, r, sl], 0.0)
                return carry
            lax.fori_loop(0, chunk, relu_row, 0)

            pltpu.sync_copy(rows_v.at[b], agg_sh.at[dsts[b]],
                            add=True)

        pltpu.sync_copy(ei_hbm.at[pl.ds(base_e, chunk)], src0)
        start_fetch(0, 0)

        def pair_body(t, carry_in):
            half_step(2 * t, 0)
            half_step(2 * t + 1, 1)
            return carry_in
        lax.fori_loop(0, n_chunks // 2, pair_body, 0)
        if n_chunks % 2:
            half_step(n_chunks - 1, 0)

        plsc.subcore_barrier()
        pltpu.sync_copy(agg_sh.at[pl.ds(row0, rows_per_tile)],
                        out_hbm.at[c, pl.ds(row0, rows_per_tile)])

    return sc_agg


def kernel(x, edge_index, edge_attr, W_msg, W_edge, b_msg, W_self, W_gate,
           W_val):
    n_nodes, d_feat = x.shape
    n_edges = edge_index.shape[1]
    d_edge = edge_attr.shape[1]
    d_hid = W_msg.shape[1]
    d_out = W_gate.shape[1]

    ei_flat = edge_index.reshape(2 * n_edges)

    bn = 2000
    xm = pl.pallas_call(
        _xm_body,
        grid=(n_nodes // bn,),
        in_specs=[pl.BlockSpec((bn, d_feat), lambda i: (i, 0)),
                  pl.BlockSpec((d_feat, d_hid), lambda i: (0, 0))],
        out_specs=pl.BlockSpec((bn, d_hid), lambda i: (i, 0)),
        out_shape=jax.ShapeDtypeStruct((n_nodes, d_hid), jnp.float32),
    )(x, W_msg)

    be = 3200
    em = pl.pallas_call(
        _em_body,
        grid=(n_edges // be,),
        in_specs=[pl.BlockSpec((d_edge, be), lambda i: (0, i)),
                  pl.BlockSpec((d_edge, d_hid), lambda i: (0, 0)),
                  pl.BlockSpec((1, d_hid), lambda i: (0, 0))],
        out_specs=pl.BlockSpec((be, d_hid), lambda i: (i, 0)),
        out_shape=jax.ShapeDtypeStruct((n_edges, d_hid), jnp.float32),
    )(edge_attr.T, W_edge, b_msg.reshape(1, d_hid))

    n_pad = -(-n_nodes // (NS * 8)) * (NS * 8)
    sc_agg = _make_sc_agg(n_pad, n_edges, d_hid, chunk=80)
    partials = sc_agg(xm, em, ei_flat)

    out = pl.pallas_call(
        _epilogue_body,
        grid=(n_nodes // bn,),
        in_specs=[pl.BlockSpec((1, bn, d_hid), lambda i: (0, i, 0)),
                  pl.BlockSpec((1, bn, d_hid), lambda i: (1, i, 0)),
                  pl.BlockSpec((bn, d_feat), lambda i: (i, 0)),
                  pl.BlockSpec((d_feat, d_hid), lambda i: (0, 0)),
                  pl.BlockSpec((d_hid, d_out), lambda i: (0, 0)),
                  pl.BlockSpec((d_hid, d_out), lambda i: (0, 0))],
        out_specs=pl.BlockSpec((bn, d_out), lambda i: (i, 0)),
        out_shape=jax.ShapeDtypeStruct((n_nodes, d_out), jnp.float32),
    )(partials, partials, x, W_self, W_gate, W_val)
    return out
